# baseline (speedup 1.0000x reference)
import functools

import numpy as np
import jax
from jax import lax
import jax.numpy as jnp
from jax.experimental import pallas as pl
from jax.experimental.pallas import tpu as pltpu
from jax.experimental.pallas import tpu_sc as plsc

_B, _H, _W = 32, 384, 384
_MINH, _MINW = 4, 4
_BLK = 8
_NC, _NS = 2, 16
_R = 96
_NV = _W // 16


def _draw_vals():
    key = jax.random.key(42)
    factor = jax.random.uniform(
        jax.random.fold_in(key, 0), (1,), minval=0.0, maxval=0.5)
    keys = jax.random.split(jax.random.fold_in(key, 1), _B)

    def per(k):
        k0, k1, k2 = jax.random.split(k, 3)
        coin = jax.random.uniform(k0, ()) < 0.5
        h_hi, h_lo = jax.random.split(k1)
        w_hi, w_lo = jax.random.split(k2)
        bits = lambda kk: jax.lax.bitcast_convert_type(
            jax.random.bits(kk, (), jnp.uint32), jnp.int32)
        return coin, bits(h_hi), bits(h_lo), bits(w_hi), bits(w_lo)

    coin, hh, hl, wh, wl = jax.vmap(per)(keys)
    tab = jnp.stack([coin.astype(jnp.int32), hh, hl, wh, wl], axis=1)
    u = jax.random.uniform(
        jax.random.fold_in(key, 2), (_B, 1, _H, _W), dtype=jnp.float32)
    return factor, tab, u.reshape(_B, _H, _W)


_CONSTS = []


def _consts():
    if not _CONSTS:
        try:
            try:
                cpu = jax.local_devices(backend="cpu")[0]
            except Exception:
                cpu = None
            if cpu is not None:
                with jax.default_device(cpu):
                    vals = jax.tree.map(np.asarray, _draw_vals())
            else:
                vals = jax.tree.map(np.asarray, _draw_vals())
            _CONSTS.append(vals)
        except Exception:
            _CONSTS.append(None)
    return _CONSTS[0]


_consts()

_ONES = np.ones((_R, _W), np.float32)


def _umod(v, span, wrap):
    r = jax.lax.rem(v, span)
    r = jnp.where(r < 0, r + span, r)
    r = r + jnp.where(v < 0, wrap, 0)
    return jnp.where(r >= span, r - span, r)


def _rand_offset(hi, lo, span):
    m16 = jax.lax.rem(jnp.int32(1 << 16), span)
    mult = jax.lax.rem(m16 * m16, span)
    hmod = _umod(hi, span, mult)
    lmod = _umod(lo, span, mult)
    return jax.lax.rem(hmod * mult + lmod, span)


def _k1_body(factor_ref, tab_ref, x_ref, itab_ref):
    bj = pl.program_id(0)
    factor = factor_ref[0]
    riota = jax.lax.broadcasted_iota(jnp.int32, (_H, 1), 0)
    ciota = jax.lax.broadcasted_iota(jnp.int32, (1, _W), 1)
    c128 = jax.lax.broadcasted_iota(jnp.int32, (1, 128), 1)
    for jj in range(_BLK):
        j = bj * _BLK + jj
        xb = x_ref[jj]

        rowmax = jnp.max(xb, axis=1, keepdims=True)
        colmax = jnp.max(xb, axis=0, keepdims=True)
        gmax = jnp.max(rowmax)
        gmin = jnp.min(xb)
        thr = gmax - (gmax - gmin) * factor

        flat = riota * _W + ciota
        center = jnp.min(jnp.where(xb == gmax, flat, _H * _W))
        cy = center // _W
        cx = center - cy * _W

        rab = rowmax > thr
        cab = colmax > thr
        miny = jnp.min(jnp.where(rab, riota, _H))
        maxy = jnp.max(jnp.where(rab, riota, -1))
        minx = jnp.min(jnp.where(cab, ciota, _W))
        maxx = jnp.max(jnp.where(cab, ciota, -1))
        max_h = maxy - miny
        max_w = maxx - minx
        valid = (max_h >= 2 * _MINH + 2) & (max_w >= 2 * _MINW + 2)

        span_h = jnp.maximum(max_h, 2 * _MINH + 2) // 2 - _MINH
        span_w = jnp.maximum(max_w, 2 * _MINW + 2) // 2 - _MINW
        h = _MINH + _rand_offset(tab_ref[j, 1], tab_ref[j, 2], span_h)
        w = _MINW + _rand_offset(tab_ref[j, 3], tab_ref[j, 4], span_w)

        h_start = jnp.maximum(cy - h, 0)
        h_end = jnp.minimum(cy + h, _W)
        w_start = jnp.maximum(cx - w, 0)
        w_end = jnp.minimum(cx + w, _W)
        erase = ((tab_ref[j, 0] > 0) & valid).astype(jnp.int32)
        tbits = jax.lax.bitcast_convert_type(thr, jnp.int32)

        rowi = jnp.where(c128 == 0, h_start, 0)
        rowi = jnp.where(c128 == 1, h_end, rowi)
        rowi = jnp.where(c128 == 2, w_start, rowi)
        rowi = jnp.where(c128 == 3, w_end, rowi)
        rowi = jnp.where(c128 == 4, erase, rowi)
        rowi = jnp.where(c128 == 5, tbits, rowi)
        itab_ref[pl.ds(jj, 1), :] = rowi


def _k2_body(itab_ref, x_ref, u_ref, out_ref):
    bj = pl.program_id(0)
    riota = jax.lax.broadcasted_iota(jnp.int32, (_H, 1), 0)
    ciota = jax.lax.broadcasted_iota(jnp.int32, (1, _W), 1)
    for jj in range(_BLK):
        j = bj * _BLK + jj
        hs = itab_ref[j, 0]
        he = itab_ref[j, 1]
        ws = itab_ref[j, 2]
        we = itab_ref[j, 3]
        er = itab_ref[j, 4] > 0
        thr = jax.lax.bitcast_convert_type(itab_ref[j, 5], jnp.float32)
        xb = x_ref[jj]
        ub = u_ref[jj]
        cond = ((xb > thr)
                & (riota > hs) & (riota < he)
                & (ciota > ws) & (ciota < we)
                & er)
        m = jnp.where(cond, 0.0, 1.0).astype(jnp.float32)
        a = 0.6 * xb + 0.2
        bern = (ub < 1.0 - a).astype(jnp.float32)
        out_ref[jj] = a * ((1.0 - m) * bern + m)


@functools.partial(
    pl.kernel,
    mesh=plsc.VectorSubcoreMesh(core_axis_name="c", subcore_axis_name="s"),
    out_type=jax.ShapeDtypeStruct((_B, _H, _W), jnp.float32),
    scratch_types=[
        pltpu.VMEM((128,), jnp.int32),
        pltpu.VMEM((_R, _W), jnp.float32),
        pltpu.VMEM((_R, _W), jnp.float32),
        pltpu.SemaphoreType.DMA,
    ],
)
def _sc_mask(x_hbm, itab_hbm, ones_hbm, mask_hbm, ti_v, ones_v, bufc_v, sem):
    wid = lax.axis_index("s") * _NC + lax.axis_index("c")
    pltpu.async_copy(ones_hbm, ones_v, sem)
    pltpu.sync_copy(itab_hbm.at[wid], ti_v)
    row_i = ti_v[pl.ds(0, 16)]

    hs = row_i[0]
    he = row_i[1]
    ws = row_i[2]
    we = row_i[3]
    erb = row_i[4] > 0
    thr = jax.lax.bitcast_convert_type(row_i[5], jnp.float32)

    colok = []
    for c in range(_NV):
        colv = lax.broadcasted_iota(jnp.int32, (16,), 0) + c * 16
        colok.append((colv > ws) & (colv < we))

    pltpu.make_async_copy(ones_hbm, ones_v, sem).wait()

    n_chunks = _H // _R
    inters = []
    for k in range(n_chunks):
        r0 = k * _R
        lo = jnp.maximum(hs + 1, r0)
        hi = jnp.minimum(he - 1, r0 + _R - 1)
        inters.append((lo <= hi) & erb)

    for k in range(n_chunks):
        r0 = k * _R

        @pl.when(jnp.logical_not(inters[k]))
        def _():
            pltpu.async_copy(ones_v, mask_hbm.at[wid, pl.ds(r0, _R)], sem)

    for k in range(n_chunks):
        r0 = k * _R

        @pl.when(inters[k])
        def _():
            pltpu.sync_copy(x_hbm.at[wid, pl.ds(r0, _R)], bufc_v)

            def row_body(r, _):
                rr = r0 + r
                rowin = (rr > hs) & (rr < he)
                thr_row = jnp.where(rowin, thr, jnp.float32(3.0e38))
                for c in range(_NV):
                    xv = bufc_v[r, pl.ds(c * 16, 16)]
                    sel = jnp.where(colok[c] & (xv > thr_row), 0.0, 1.0)
                    bufc_v[r, pl.ds(c * 16, 16)] = sel.astype(jnp.float32)
                return 0

            lax.fori_loop(0, _R, row_body, 0)
            pltpu.sync_copy(bufc_v, mask_hbm.at[wid, pl.ds(r0, _R)])

    for k in range(n_chunks):
        r0 = k * _R

        @pl.when(jnp.logical_not(inters[k]))
        def _():
            pltpu.make_async_copy(
                ones_v, mask_hbm.at[wid, pl.ds(r0, _R)], sem).wait()


@jax.jit
def _run(x3, factor, tab, u3, ones):
    itab = pl.pallas_call(
        _k1_body,
        grid=(_B // _BLK,),
        in_specs=[
            pl.BlockSpec(memory_space=pltpu.SMEM),
            pl.BlockSpec(memory_space=pltpu.SMEM),
            pl.BlockSpec((_BLK, _H, _W), lambda i: (i, 0, 0)),
        ],
        out_specs=pl.BlockSpec((_BLK, 128), lambda i: (i, 0)),
        out_shape=jax.ShapeDtypeStruct((_B, 128), jnp.int32),
    )(factor, tab, x3)

    mask3 = _sc_mask(x3, itab, ones)

    out3 = pl.pallas_call(
        _k2_body,
        grid=(_B // _BLK,),
        in_specs=[
            pl.BlockSpec(memory_space=pltpu.SMEM),
            pl.BlockSpec((_BLK, _H, _W), lambda i: (i, 0, 0)),
            pl.BlockSpec((_BLK, _H, _W), lambda i: (i, 0, 0)),
        ],
        out_specs=pl.BlockSpec((_BLK, _H, _W), lambda i: (i, 0, 0)),
        out_shape=jax.ShapeDtypeStruct((_B, _H, _W), jnp.float32),
    )(itab, x3, u3)
    return out3, mask3


def kernel(x):
    c = _consts()
    factor, tab, u3 = c if c is not None else _draw_vals()
    out3, mask3 = _run(x.reshape(_B, _H, _W), factor, tab, u3, _ONES)
    return out3.reshape(_B, 1, _H, _W), mask3.reshape(_B, 1, _H, _W)


# speedup vs baseline: 1.0293x; 1.0293x over previous
import functools

import numpy as np
import jax
from jax import lax
import jax.numpy as jnp
from jax.experimental import pallas as pl
from jax.experimental.pallas import tpu as pltpu
from jax.experimental.pallas import tpu_sc as plsc

_B, _H, _W = 32, 384, 384
_MINH, _MINW = 4, 4
_BLK = 8
_NC, _NS = 2, 16
_R = 48
_NV = _W // 16


def _draw_vals():
    key = jax.random.key(42)
    factor = jax.random.uniform(
        jax.random.fold_in(key, 0), (1,), minval=0.0, maxval=0.5)
    keys = jax.random.split(jax.random.fold_in(key, 1), _B)

    def per(k):
        k0, k1, k2 = jax.random.split(k, 3)
        coin = jax.random.uniform(k0, ()) < 0.5
        h_hi, h_lo = jax.random.split(k1)
        w_hi, w_lo = jax.random.split(k2)
        bits = lambda kk: jax.lax.bitcast_convert_type(
            jax.random.bits(kk, (), jnp.uint32), jnp.int32)
        return coin, bits(h_hi), bits(h_lo), bits(w_hi), bits(w_lo)

    coin, hh, hl, wh, wl = jax.vmap(per)(keys)
    tab = jnp.stack([coin.astype(jnp.int32), hh, hl, wh, wl], axis=1)
    u = jax.random.uniform(
        jax.random.fold_in(key, 2), (_B, 1, _H, _W), dtype=jnp.float32)
    return factor, tab, u.reshape(_B, _H, _W)


_CONSTS = []


def _consts():
    if not _CONSTS:
        try:
            try:
                cpu = jax.local_devices(backend="cpu")[0]
            except Exception:
                cpu = None
            if cpu is not None:
                with jax.default_device(cpu):
                    vals = jax.tree.map(np.asarray, _draw_vals())
            else:
                vals = jax.tree.map(np.asarray, _draw_vals())
            _CONSTS.append(vals)
        except Exception:
            _CONSTS.append(None)
    return _CONSTS[0]


_consts()

_ONES = np.ones((_R, _W), np.float32)


def _umod(v, span, wrap):
    r = jax.lax.rem(v, span)
    r = jnp.where(r < 0, r + span, r)
    r = r + jnp.where(v < 0, wrap, 0)
    return jnp.where(r >= span, r - span, r)


def _rand_offset(hi, lo, span):
    m16 = jax.lax.rem(jnp.int32(1 << 16), span)
    mult = jax.lax.rem(m16 * m16, span)
    hmod = _umod(hi, span, mult)
    lmod = _umod(lo, span, mult)
    return jax.lax.rem(hmod * mult + lmod, span)


def _k1_body(factor_ref, tab_ref, x_ref, itab_ref):
    bj = pl.program_id(0)
    factor = factor_ref[0]
    riota = jax.lax.broadcasted_iota(jnp.int32, (_H, 1), 0)
    ciota = jax.lax.broadcasted_iota(jnp.int32, (1, _W), 1)
    c128 = jax.lax.broadcasted_iota(jnp.int32, (1, 128), 1)
    for jj in range(_BLK):
        j = bj * _BLK + jj
        xb = x_ref[jj]

        rowmax = jnp.max(xb, axis=1, keepdims=True)
        colmax = jnp.max(xb, axis=0, keepdims=True)
        gmax = jnp.max(rowmax)
        gmin = jnp.min(xb)
        thr = gmax - (gmax - gmin) * factor

        flat = riota * _W + ciota
        center = jnp.min(jnp.where(xb == gmax, flat, _H * _W))
        cy = center // _W
        cx = center - cy * _W

        rab = rowmax > thr
        cab = colmax > thr
        miny = jnp.min(jnp.where(rab, riota, _H))
        maxy = jnp.max(jnp.where(rab, riota, -1))
        minx = jnp.min(jnp.where(cab, ciota, _W))
        maxx = jnp.max(jnp.where(cab, ciota, -1))
        max_h = maxy - miny
        max_w = maxx - minx
        valid = (max_h >= 2 * _MINH + 2) & (max_w >= 2 * _MINW + 2)

        span_h = jnp.maximum(max_h, 2 * _MINH + 2) // 2 - _MINH
        span_w = jnp.maximum(max_w, 2 * _MINW + 2) // 2 - _MINW
        h = _MINH + _rand_offset(tab_ref[j, 1], tab_ref[j, 2], span_h)
        w = _MINW + _rand_offset(tab_ref[j, 3], tab_ref[j, 4], span_w)

        h_start = jnp.maximum(cy - h, 0)
        h_end = jnp.minimum(cy + h, _W)
        w_start = jnp.maximum(cx - w, 0)
        w_end = jnp.minimum(cx + w, _W)
        erase = ((tab_ref[j, 0] > 0) & valid).astype(jnp.int32)
        tbits = jax.lax.bitcast_convert_type(thr, jnp.int32)

        rowi = jnp.where(c128 == 0, h_start, 0)
        rowi = jnp.where(c128 == 1, h_end, rowi)
        rowi = jnp.where(c128 == 2, w_start, rowi)
        rowi = jnp.where(c128 == 3, w_end, rowi)
        rowi = jnp.where(c128 == 4, erase, rowi)
        rowi = jnp.where(c128 == 5, tbits, rowi)
        itab_ref[pl.ds(jj, 1), :] = rowi


def _k2_body(itab_ref, x_ref, u_ref, out_ref):
    bj = pl.program_id(0)
    riota = jax.lax.broadcasted_iota(jnp.int32, (_H, 1), 0)
    ciota = jax.lax.broadcasted_iota(jnp.int32, (1, _W), 1)
    for jj in range(_BLK):
        j = bj * _BLK + jj
        hs = itab_ref[j, 0]
        he = itab_ref[j, 1]
        ws = itab_ref[j, 2]
        we = itab_ref[j, 3]
        er = itab_ref[j, 4] > 0
        thr = jax.lax.bitcast_convert_type(itab_ref[j, 5], jnp.float32)
        xb = x_ref[jj]
        ub = u_ref[jj]
        cond = ((xb > thr)
                & (riota > hs) & (riota < he)
                & (ciota > ws) & (ciota < we)
                & er)
        m = jnp.where(cond, 0.0, 1.0).astype(jnp.float32)
        a = 0.6 * xb + 0.2
        bern = (ub < 1.0 - a).astype(jnp.float32)
        out_ref[jj] = a * ((1.0 - m) * bern + m)


@functools.partial(
    pl.kernel,
    mesh=plsc.VectorSubcoreMesh(core_axis_name="c", subcore_axis_name="s"),
    out_type=jax.ShapeDtypeStruct((_B, _H, _W), jnp.float32),
    scratch_types=[
        pltpu.VMEM((128,), jnp.int32),
        pltpu.VMEM((_R, _W), jnp.float32),
        pltpu.VMEM((_R, _W), jnp.float32),
        pltpu.SemaphoreType.DMA,
    ],
)
def _sc_mask(x_hbm, itab_hbm, ones_hbm, mask_hbm, ti_v, ones_v, bufc_v, sem):
    wid = lax.axis_index("s") * _NC + lax.axis_index("c")
    pltpu.async_copy(ones_hbm, ones_v, sem)
    pltpu.sync_copy(itab_hbm.at[wid], ti_v)
    row_i = ti_v[pl.ds(0, 16)]

    hs = row_i[0]
    he = row_i[1]
    ws = row_i[2]
    we = row_i[3]
    erb = row_i[4] > 0
    thr = jax.lax.bitcast_convert_type(row_i[5], jnp.float32)

    colok = []
    for c in range(_NV):
        colv = lax.broadcasted_iota(jnp.int32, (16,), 0) + c * 16
        colok.append((colv > ws) & (colv < we))

    pltpu.make_async_copy(ones_hbm, ones_v, sem).wait()

    n_chunks = _H // _R
    inters = []
    for k in range(n_chunks):
        r0 = k * _R
        lo = jnp.maximum(hs + 1, r0)
        hi = jnp.minimum(he - 1, r0 + _R - 1)
        inters.append((lo <= hi) & erb)

    for k in range(n_chunks):
        r0 = k * _R

        @pl.when(jnp.logical_not(inters[k]))
        def _():
            pltpu.async_copy(ones_v, mask_hbm.at[wid, pl.ds(r0, _R)], sem)

    for k in range(n_chunks):
        r0 = k * _R

        @pl.when(inters[k])
        def _():
            pltpu.sync_copy(x_hbm.at[wid, pl.ds(r0, _R)], bufc_v)

            def row_body(r, _):
                rr = r0 + r
                rowin = (rr > hs) & (rr < he)
                thr_row = jnp.where(rowin, thr, jnp.float32(3.0e38))
                for c in range(_NV):
                    xv = bufc_v[r, pl.ds(c * 16, 16)]
                    sel = jnp.where(colok[c] & (xv > thr_row), 0.0, 1.0)
                    bufc_v[r, pl.ds(c * 16, 16)] = sel.astype(jnp.float32)
                return 0

            lax.fori_loop(0, _R, row_body, 0)
            pltpu.sync_copy(bufc_v, mask_hbm.at[wid, pl.ds(r0, _R)])

    for k in range(n_chunks):
        r0 = k * _R

        @pl.when(jnp.logical_not(inters[k]))
        def _():
            pltpu.make_async_copy(
                ones_v, mask_hbm.at[wid, pl.ds(r0, _R)], sem).wait()


@jax.jit
def _run(x3, factor, tab, u3, ones):
    itab = pl.pallas_call(
        _k1_body,
        grid=(_B // _BLK,),
        in_specs=[
            pl.BlockSpec(memory_space=pltpu.SMEM),
            pl.BlockSpec(memory_space=pltpu.SMEM),
            pl.BlockSpec((_BLK, _H, _W), lambda i: (i, 0, 0)),
        ],
        out_specs=pl.BlockSpec((_BLK, 128), lambda i: (i, 0)),
        out_shape=jax.ShapeDtypeStruct((_B, 128), jnp.int32),
    )(factor, tab, x3)

    mask3 = _sc_mask(x3, itab, ones)

    out3 = pl.pallas_call(
        _k2_body,
        grid=(_B // _BLK,),
        in_specs=[
            pl.BlockSpec(memory_space=pltpu.SMEM),
            pl.BlockSpec((_BLK, _H, _W), lambda i: (i, 0, 0)),
            pl.BlockSpec((_BLK, _H, _W), lambda i: (i, 0, 0)),
        ],
        out_specs=pl.BlockSpec((_BLK, _H, _W), lambda i: (i, 0, 0)),
        out_shape=jax.ShapeDtypeStruct((_B, _H, _W), jnp.float32),
    )(itab, x3, u3)
    return out3, mask3


def kernel(x):
    c = _consts()
    factor, tab, u3 = c if c is not None else _draw_vals()
    out3, mask3 = _run(x.reshape(_B, _H, _W), factor, tab, u3, _ONES)
    return out3.reshape(_B, 1, _H, _W), mask3.reshape(_B, 1, _H, _W)


# speedup vs baseline: 1.0490x; 1.0192x over previous
import functools

import numpy as np
import jax
from jax import lax
import jax.numpy as jnp
from jax.experimental import pallas as pl
from jax.experimental.pallas import tpu as pltpu
from jax.experimental.pallas import tpu_sc as plsc

_B, _H, _W = 32, 384, 384
_MINH, _MINW = 4, 4
_BLK = 8
_NC, _NS = 2, 16
_R = 48
_NV = _W // 16


def _draw_vals():
    key = jax.random.key(42)
    factor = jax.random.uniform(
        jax.random.fold_in(key, 0), (1,), minval=0.0, maxval=0.5)
    keys = jax.random.split(jax.random.fold_in(key, 1), _B)

    def per(k):
        k0, k1, k2 = jax.random.split(k, 3)
        coin = jax.random.uniform(k0, ()) < 0.5
        h_hi, h_lo = jax.random.split(k1)
        w_hi, w_lo = jax.random.split(k2)
        bits = lambda kk: jax.lax.bitcast_convert_type(
            jax.random.bits(kk, (), jnp.uint32), jnp.int32)
        return coin, bits(h_hi), bits(h_lo), bits(w_hi), bits(w_lo)

    coin, hh, hl, wh, wl = jax.vmap(per)(keys)
    tab = jnp.stack([coin.astype(jnp.int32), hh, hl, wh, wl], axis=1)
    u = jax.random.uniform(
        jax.random.fold_in(key, 2), (_B, 1, _H, _W), dtype=jnp.float32)
    return factor, tab, u.reshape(_B, _H, _W)


_CONSTS = []


def _consts():
    if not _CONSTS:
        try:
            try:
                cpu = jax.local_devices(backend="cpu")[0]
            except Exception:
                cpu = None
            if cpu is not None:
                with jax.default_device(cpu):
                    vals = jax.tree.map(np.asarray, _draw_vals())
            else:
                vals = jax.tree.map(np.asarray, _draw_vals())
            _CONSTS.append(vals)
        except Exception:
            _CONSTS.append(None)
    return _CONSTS[0]


_consts()

_ONES = np.ones((_R, _W), np.float32)


def _umod(v, span, wrap):
    r = jax.lax.rem(v, span)
    r = jnp.where(r < 0, r + span, r)
    r = r + jnp.where(v < 0, wrap, 0)
    return jnp.where(r >= span, r - span, r)


def _rand_offset(hi, lo, span):
    m16 = jax.lax.rem(jnp.int32(1 << 16), span)
    mult = jax.lax.rem(m16 * m16, span)
    hmod = _umod(hi, span, mult)
    lmod = _umod(lo, span, mult)
    return jax.lax.rem(hmod * mult + lmod, span)


def _k1_body(factor_ref, tab_ref, x_ref, itab_ref):
    bj = pl.program_id(0)
    factor = factor_ref[0]
    riota = jax.lax.broadcasted_iota(jnp.int32, (_H, 1), 0)
    ciota = jax.lax.broadcasted_iota(jnp.int32, (1, _W), 1)
    c128 = jax.lax.broadcasted_iota(jnp.int32, (1, 128), 1)
    for jj in range(_BLK):
        j = bj * _BLK + jj
        xb = x_ref[jj]

        rowmax = jnp.max(xb, axis=1, keepdims=True)
        colmax = jnp.max(xb, axis=0, keepdims=True)
        gmax = jnp.max(rowmax)
        gmin = jnp.min(xb)
        thr = gmax - (gmax - gmin) * factor

        flat = riota * _W + ciota
        center = jnp.min(jnp.where(xb == gmax, flat, _H * _W))
        cy = center // _W
        cx = center - cy * _W

        rab = rowmax > thr
        cab = colmax > thr
        miny = jnp.min(jnp.where(rab, riota, _H))
        maxy = jnp.max(jnp.where(rab, riota, -1))
        minx = jnp.min(jnp.where(cab, ciota, _W))
        maxx = jnp.max(jnp.where(cab, ciota, -1))
        max_h = maxy - miny
        max_w = maxx - minx
        valid = (max_h >= 2 * _MINH + 2) & (max_w >= 2 * _MINW + 2)

        span_h = jnp.maximum(max_h, 2 * _MINH + 2) // 2 - _MINH
        span_w = jnp.maximum(max_w, 2 * _MINW + 2) // 2 - _MINW
        h = _MINH + _rand_offset(tab_ref[j, 1], tab_ref[j, 2], span_h)
        w = _MINW + _rand_offset(tab_ref[j, 3], tab_ref[j, 4], span_w)

        h_start = jnp.maximum(cy - h, 0)
        h_end = jnp.minimum(cy + h, _W)
        w_start = jnp.maximum(cx - w, 0)
        w_end = jnp.minimum(cx + w, _W)
        erase = ((tab_ref[j, 0] > 0) & valid).astype(jnp.int32)
        tbits = jax.lax.bitcast_convert_type(thr, jnp.int32)

        rowi = jnp.where(c128 == 0, h_start, 0)
        rowi = jnp.where(c128 == 1, h_end, rowi)
        rowi = jnp.where(c128 == 2, w_start, rowi)
        rowi = jnp.where(c128 == 3, w_end, rowi)
        rowi = jnp.where(c128 == 4, erase, rowi)
        rowi = jnp.where(c128 == 5, tbits, rowi)
        itab_ref[pl.ds(jj, 1), :] = rowi


def _k2_body(itab_ref, x_ref, u_ref, out_ref):
    bj = pl.program_id(0)
    riota = jax.lax.broadcasted_iota(jnp.int32, (_H, 1), 0)
    ciota = jax.lax.broadcasted_iota(jnp.int32, (1, _W), 1)
    for jj in range(_BLK):
        j = bj * _BLK + jj
        hs = itab_ref[j, 0]
        he = itab_ref[j, 1]
        ws = itab_ref[j, 2]
        we = itab_ref[j, 3]
        er = itab_ref[j, 4] > 0
        thr = jax.lax.bitcast_convert_type(itab_ref[j, 5], jnp.float32)
        xb = x_ref[jj]
        ub = u_ref[jj]
        cond = ((xb > thr)
                & (riota > hs) & (riota < he)
                & (ciota > ws) & (ciota < we)
                & er)
        m = jnp.where(cond, 0.0, 1.0).astype(jnp.float32)
        a = 0.6 * xb + 0.2
        bern = (ub < 1.0 - a).astype(jnp.float32)
        out_ref[jj] = a * ((1.0 - m) * bern + m)


@functools.partial(
    pl.kernel,
    mesh=plsc.VectorSubcoreMesh(core_axis_name="c", subcore_axis_name="s"),
    out_type=jax.ShapeDtypeStruct((_B, _H, _W), jnp.float32),
    scratch_types=[
        pltpu.VMEM((128,), jnp.int32),
        pltpu.VMEM((_R, _W), jnp.float32),
        pltpu.VMEM((_R, _W), jnp.float32),
        pltpu.VMEM((_R, _W), jnp.float32),
        pltpu.SemaphoreType.DMA,
    ],
)
def _sc_mask(x_hbm, itab_hbm, ones_hbm, mask_hbm, ti_v, ones_v, bufx_v,
             bufm_v, sem):
    wid = lax.axis_index("s") * _NC + lax.axis_index("c")
    pltpu.async_copy(ones_hbm, ones_v, sem)
    pltpu.sync_copy(itab_hbm.at[wid], ti_v)
    row_i = ti_v[pl.ds(0, 16)]

    hs = row_i[0]
    he = row_i[1]
    ws = row_i[2]
    we = row_i[3]
    erb = row_i[4] > 0
    thr = jax.lax.bitcast_convert_type(row_i[5], jnp.float32)

    pltpu.make_async_copy(ones_hbm, ones_v, sem).wait()

    n_chunks = _H // _R
    inters = []
    for k in range(n_chunks):
        r0 = k * _R
        lo = jnp.maximum(hs + 1, r0)
        hi = jnp.minimum(he - 1, r0 + _R - 1)
        inters.append((lo <= hi) & erb)

    for k in range(n_chunks):
        r0 = k * _R

        @pl.when(jnp.logical_not(inters[k]))
        def _():
            pltpu.async_copy(ones_v, mask_hbm.at[wid, pl.ds(r0, _R)], sem)

    for k in range(n_chunks):
        r0 = k * _R

        @pl.when(inters[k])
        def _():
            pltpu.sync_copy(x_hbm.at[wid, pl.ds(r0, _R)], bufx_v)

            def row_body(r, _):
                rr = r0 + r
                rowin = (rr > hs) & (rr < he)
                thr_row = jnp.where(rowin, thr, jnp.float32(3.0e38))
                for c in range(_NV):
                    colv = lax.broadcasted_iota(jnp.int32, (16,), 0) + c * 16
                    colok = (colv > ws) & (colv < we)
                    xv = bufx_v[r, pl.ds(c * 16, 16)]
                    sel = jnp.where(colok & (xv > thr_row), 0.0, 1.0)
                    bufm_v[r, pl.ds(c * 16, 16)] = sel.astype(jnp.float32)
                return 0

            lax.fori_loop(0, _R, row_body, 0)
            pltpu.sync_copy(bufm_v, mask_hbm.at[wid, pl.ds(r0, _R)])

    for k in range(n_chunks):
        r0 = k * _R

        @pl.when(jnp.logical_not(inters[k]))
        def _():
            pltpu.make_async_copy(
                ones_v, mask_hbm.at[wid, pl.ds(r0, _R)], sem).wait()


@jax.jit
def _run(x3, factor, tab, u3, ones):
    itab = pl.pallas_call(
        _k1_body,
        grid=(_B // _BLK,),
        in_specs=[
            pl.BlockSpec(memory_space=pltpu.SMEM),
            pl.BlockSpec(memory_space=pltpu.SMEM),
            pl.BlockSpec((_BLK, _H, _W), lambda i: (i, 0, 0)),
        ],
        out_specs=pl.BlockSpec((_BLK, 128), lambda i: (i, 0)),
        out_shape=jax.ShapeDtypeStruct((_B, 128), jnp.int32),
    )(factor, tab, x3)

    mask3 = _sc_mask(x3, itab, ones)

    out3 = pl.pallas_call(
        _k2_body,
        grid=(_B // _BLK,),
        in_specs=[
            pl.BlockSpec(memory_space=pltpu.SMEM),
            pl.BlockSpec((_BLK, _H, _W), lambda i: (i, 0, 0)),
            pl.BlockSpec((_BLK, _H, _W), lambda i: (i, 0, 0)),
        ],
        out_specs=pl.BlockSpec((_BLK, _H, _W), lambda i: (i, 0, 0)),
        out_shape=jax.ShapeDtypeStruct((_B, _H, _W), jnp.float32),
    )(itab, x3, u3)
    return out3, mask3


def kernel(x):
    c = _consts()
    factor, tab, u3 = c if c is not None else _draw_vals()
    out3, mask3 = _run(x.reshape(_B, _H, _W), factor, tab, u3, _ONES)
    return out3.reshape(_B, 1, _H, _W), mask3.reshape(_B, 1, _H, _W)


# speedup vs baseline: 1.0916x; 1.0406x over previous
import functools

import numpy as np
import jax
from jax import lax
import jax.numpy as jnp
from jax.experimental import pallas as pl
from jax.experimental.pallas import tpu as pltpu
from jax.experimental.pallas import tpu_sc as plsc

_B, _H, _W = 32, 384, 384
_MINH, _MINW = 4, 4
_BLK = 8
_NC, _NS = 2, 16
_R = 48
_NV = _W // 16


def _draw_vals():
    key = jax.random.key(42)
    factor = jax.random.uniform(
        jax.random.fold_in(key, 0), (1,), minval=0.0, maxval=0.5)
    keys = jax.random.split(jax.random.fold_in(key, 1), _B)

    def per(k):
        k0, k1, k2 = jax.random.split(k, 3)
        coin = jax.random.uniform(k0, ()) < 0.5
        h_hi, h_lo = jax.random.split(k1)
        w_hi, w_lo = jax.random.split(k2)
        bits = lambda kk: jax.lax.bitcast_convert_type(
            jax.random.bits(kk, (), jnp.uint32), jnp.int32)
        return coin, bits(h_hi), bits(h_lo), bits(w_hi), bits(w_lo)

    coin, hh, hl, wh, wl = jax.vmap(per)(keys)
    tab = jnp.stack([coin.astype(jnp.int32), hh, hl, wh, wl], axis=1)
    u = jax.random.uniform(
        jax.random.fold_in(key, 2), (_B, 1, _H, _W), dtype=jnp.float32)
    return factor, tab, u.reshape(_B, _H, _W)


_CONSTS = []


def _consts():
    if not _CONSTS:
        try:
            try:
                cpu = jax.local_devices(backend="cpu")[0]
            except Exception:
                cpu = None
            if cpu is not None:
                with jax.default_device(cpu):
                    vals = jax.tree.map(np.asarray, _draw_vals())
            else:
                vals = jax.tree.map(np.asarray, _draw_vals())
            _CONSTS.append(vals)
        except Exception:
            _CONSTS.append(None)
    return _CONSTS[0]


_consts()


def _umod(v, span, wrap):
    r = jax.lax.rem(v, span)
    r = jnp.where(r < 0, r + span, r)
    r = r + jnp.where(v < 0, wrap, 0)
    return jnp.where(r >= span, r - span, r)


def _rand_offset(hi, lo, span):
    m16 = jax.lax.rem(jnp.int32(1 << 16), span)
    mult = jax.lax.rem(m16 * m16, span)
    hmod = _umod(hi, span, mult)
    lmod = _umod(lo, span, mult)
    return jax.lax.rem(hmod * mult + lmod, span)


def _k1_body(factor_ref, tab_ref, x_ref, ftab_ref, itab_ref):
    bj = pl.program_id(0)
    factor = factor_ref[0]
    riota = jax.lax.broadcasted_iota(jnp.int32, (_H, 1), 0)
    ciota = jax.lax.broadcasted_iota(jnp.int32, (1, _W), 1)
    c128 = jax.lax.broadcasted_iota(jnp.int32, (1, 128), 1)
    for jj in range(_BLK):
        j = bj * _BLK + jj
        xb = x_ref[jj]

        rowmax = jnp.max(xb, axis=1, keepdims=True)
        colmax = jnp.max(xb, axis=0, keepdims=True)
        gmax = jnp.max(rowmax)
        gmin = jnp.min(xb)
        thr = gmax - (gmax - gmin) * factor

        flat = riota * _W + ciota
        center = jnp.min(jnp.where(xb == gmax, flat, _H * _W))
        cy = center // _W
        cx = center - cy * _W

        rab = rowmax > thr
        cab = colmax > thr
        miny = jnp.min(jnp.where(rab, riota, _H))
        maxy = jnp.max(jnp.where(rab, riota, -1))
        minx = jnp.min(jnp.where(cab, ciota, _W))
        maxx = jnp.max(jnp.where(cab, ciota, -1))
        max_h = maxy - miny
        max_w = maxx - minx
        valid = (max_h >= 2 * _MINH + 2) & (max_w >= 2 * _MINW + 2)

        span_h = jnp.maximum(max_h, 2 * _MINH + 2) // 2 - _MINH
        span_w = jnp.maximum(max_w, 2 * _MINW + 2) // 2 - _MINW
        h = _MINH + _rand_offset(tab_ref[j, 1], tab_ref[j, 2], span_h)
        w = _MINW + _rand_offset(tab_ref[j, 3], tab_ref[j, 4], span_w)

        h_start = jnp.maximum(cy - h, 0)
        h_end = jnp.minimum(cy + h, _W)
        w_start = jnp.maximum(cx - w, 0)
        w_end = jnp.minimum(cx + w, _W)
        erase = ((tab_ref[j, 0] > 0) & valid).astype(jnp.int32)

        rowf = jnp.where(c128 == 0, thr, 0.0).astype(jnp.float32)
        ftab_ref[pl.ds(jj, 1), :] = rowf

        rowi = jnp.where(c128 == 0, h_start, 0)
        rowi = jnp.where(c128 == 1, h_end, rowi)
        rowi = jnp.where(c128 == 2, w_start, rowi)
        rowi = jnp.where(c128 == 3, w_end, rowi)
        rowi = jnp.where(c128 == 4, erase, rowi)
        itab_ref[pl.ds(jj, 1), :] = rowi


def _k2_body(ftab_ref, itab_ref, x_ref, u_ref, out_ref):
    bj = pl.program_id(0)
    riota = jax.lax.broadcasted_iota(jnp.int32, (_H, 1), 0)
    ciota = jax.lax.broadcasted_iota(jnp.int32, (1, _W), 1)
    for jj in range(_BLK):
        j = bj * _BLK + jj
        thr = ftab_ref[j, 0]
        hs = itab_ref[j, 0]
        he = itab_ref[j, 1]
        ws = itab_ref[j, 2]
        we = itab_ref[j, 3]
        er = itab_ref[j, 4] > 0
        xb = x_ref[jj]
        ub = u_ref[jj]
        cond = ((xb > thr)
                & (riota > hs) & (riota < he)
                & (ciota > ws) & (ciota < we)
                & er)
        m = jnp.where(cond, 0.0, 1.0).astype(jnp.float32)
        a = 0.6 * xb + 0.2
        bern = (ub < 1.0 - a).astype(jnp.float32)
        out_ref[jj] = a * ((1.0 - m) * bern + m)


@functools.partial(
    pl.kernel,
    mesh=plsc.VectorSubcoreMesh(core_axis_name="c", subcore_axis_name="s"),
    out_type=jax.ShapeDtypeStruct((_B, _H, _W), jnp.float32),
    scratch_types=[
        pltpu.VMEM((128,), jnp.int32),
        pltpu.VMEM((128,), jnp.float32),
        pltpu.VMEM((_R, _W), jnp.float32),
        pltpu.VMEM((_R, _W), jnp.float32),
        pltpu.VMEM((_R, _W), jnp.float32),
    ],
)
def _sc_mask(x_hbm, itab_hbm, ftab_hbm, mask_hbm, ti_v, tf_v,
             ones_v, bufx_v, bufm_v):
    wid = lax.axis_index("s") * _NC + lax.axis_index("c")
    pltpu.sync_copy(itab_hbm.at[wid], ti_v)
    pltpu.sync_copy(ftab_hbm.at[wid], tf_v)
    row_i = ti_v[pl.ds(0, 16)]
    row_f = tf_v[pl.ds(0, 16)]

    hs = row_i[0]
    he = row_i[1]
    ws = row_i[2]
    we = row_i[3]
    erb = row_i[4] > 0
    thr = row_f[0]

    def fill_row(r, _):
        for c in range(_NV):
            ones_v[r, pl.ds(c * 16, 16)] = jnp.full((16,), 1.0, jnp.float32)
        return 0

    lax.fori_loop(0, _R, fill_row, 0)

    for k in range(_H // _R):
        r0 = k * _R
        lo = jnp.maximum(hs + 1, r0)
        hi = jnp.minimum(he - 1, r0 + _R - 1)
        inter = (lo <= hi) & erb

        @pl.when(inter)
        def _():
            pltpu.sync_copy(x_hbm.at[wid, pl.ds(r0, _R)], bufx_v)

            def row_body(r, _):
                rr = r0 + r
                rowin = (rr > hs) & (rr < he)
                thr_row = jnp.where(rowin, thr, jnp.float32(3.0e38))
                for c in range(_NV):
                    colv = lax.broadcasted_iota(jnp.int32, (16,), 0) + c * 16
                    colok = (colv > ws) & (colv < we)
                    xv = bufx_v[r, pl.ds(c * 16, 16)]
                    sel = jnp.where(colok & (xv > thr_row), 0.0, 1.0)
                    bufm_v[r, pl.ds(c * 16, 16)] = sel.astype(jnp.float32)
                return 0

            lax.fori_loop(0, _R, row_body, 0)
            pltpu.sync_copy(bufm_v, mask_hbm.at[wid, pl.ds(r0, _R)])

        @pl.when(jnp.logical_not(inter))
        def _():
            pltpu.sync_copy(ones_v, mask_hbm.at[wid, pl.ds(r0, _R)])


@jax.jit
def _run(x3, factor, tab, u3):
    ftab, itab = pl.pallas_call(
        _k1_body,
        grid=(_B // _BLK,),
        in_specs=[
            pl.BlockSpec(memory_space=pltpu.SMEM),
            pl.BlockSpec(memory_space=pltpu.SMEM),
            pl.BlockSpec((_BLK, _H, _W), lambda i: (i, 0, 0)),
        ],
        out_specs=[
            pl.BlockSpec((_BLK, 128), lambda i: (i, 0)),
            pl.BlockSpec((_BLK, 128), lambda i: (i, 0)),
        ],
        out_shape=[
            jax.ShapeDtypeStruct((_B, 128), jnp.float32),
            jax.ShapeDtypeStruct((_B, 128), jnp.int32),
        ],
    )(factor, tab, x3)

    mask3 = _sc_mask(x3, itab, ftab)

    out3 = pl.pallas_call(
        _k2_body,
        grid=(_B // _BLK,),
        in_specs=[
            pl.BlockSpec(memory_space=pltpu.SMEM),
            pl.BlockSpec(memory_space=pltpu.SMEM),
            pl.BlockSpec((_BLK, _H, _W), lambda i: (i, 0, 0)),
            pl.BlockSpec((_BLK, _H, _W), lambda i: (i, 0, 0)),
        ],
        out_specs=pl.BlockSpec((_BLK, _H, _W), lambda i: (i, 0, 0)),
        out_shape=jax.ShapeDtypeStruct((_B, _H, _W), jnp.float32),
    )(ftab, itab, x3, u3)
    return out3, mask3


def kernel(x):
    c = _consts()
    factor, tab, u3 = c if c is not None else _draw_vals()
    out3, mask3 = _run(x.reshape(_B, _H, _W), factor, tab, u3)
    return out3.reshape(_B, 1, _H, _W), mask3.reshape(_B, 1, _H, _W)
